# baseline (speedup 1.0000x reference)
import functools

import jax
import jax.numpy as jnp
from jax import lax
from jax.experimental import pallas as pl
from jax.experimental.pallas import tpu as pltpu
from jax.experimental.pallas import tpu_sc as plsc

E = 160000
D = 128
H = 256
ORB = 13
OO = ORB * ORB
NUM_EXPERTS = 4
BOUNDS = (1.2, 1.6, 2.0)

TM = 8000

NC = 2
NS = 16
NW = NC * NS
EPW = E // NW
EPW_PAD = 5008
CHUNKS = (EPW + 15) // 16


@functools.partial(
    pl.kernel,
    out_type=jax.ShapeDtypeStruct((E,), jnp.float32),
    mesh=plsc.VectorSubcoreMesh(core_axis_name="c", subcore_axis_name="s"),
    scratch_types=[
        pltpu.VMEM((EPW_PAD,), jnp.float32),
        pltpu.VMEM((EPW_PAD,), jnp.float32),
        pltpu.VMEM((EPW_PAD,), jnp.float32),
        pltpu.VMEM((EPW_PAD,), jnp.float32),
    ],
)
def _route_sc(x_hbm, y_hbm, z_hbm, out_hbm, vx, vy, vz, vout):
    wid = lax.axis_index("s") * NC + lax.axis_index("c")
    base = wid * EPW
    pltpu.sync_copy(x_hbm.at[pl.ds(base, EPW)], vx.at[pl.ds(0, EPW)])
    pltpu.sync_copy(y_hbm.at[pl.ds(base, EPW)], vy.at[pl.ds(0, EPW)])
    pltpu.sync_copy(z_hbm.at[pl.ds(base, EPW)], vz.at[pl.ds(0, EPW)])

    def step(c, carry):
        s = c * 16
        x = vx[pl.ds(s, 16)]
        y = vy[pl.ds(s, 16)]
        z = vz[pl.ds(s, 16)]
        d2 = x * x + y * y + z * z
        one = jnp.full((16,), 1.0, jnp.float32)
        zero = jnp.full((16,), 0.0, jnp.float32)
        e = jnp.where(d2 >= BOUNDS[0] * BOUNDS[0], one, zero)
        for b in BOUNDS[1:]:
            e = e + jnp.where(d2 >= b * b, one, zero)
        vout[pl.ds(s, 16)] = e
        return carry

    lax.fori_loop(0, CHUNKS, step, 0)
    pltpu.sync_copy(vout.at[pl.ds(0, EPW)], out_hbm.at[pl.ds(base, EPW)])


def _fused_body(idx_ref, feat_ref, w1_ref, b1_ref, w2_ref, b2_ref, out_ref):
    route = idx_ref[...]
    feat = feat_ref[...].astype(jnp.bfloat16)

    res = None
    for i in range(NUM_EXPERTS):
        h = jnp.maximum(
            jnp.dot(feat, w1_ref[i], preferred_element_type=jnp.float32)
            + b1_ref[i][None, :], 0.0).astype(jnp.bfloat16)
        o = (jnp.dot(h, w2_ref[i], preferred_element_type=jnp.float32)
             + b2_ref[i][None, :])
        if i == 0:
            res = o
        else:
            res = jnp.where(route == float(i), o, res)
    out_ref[...] = res.astype(jnp.bfloat16)


def kernel(edge_vec, edge_feat, W1, b1, W2, b2):
    route = _route_sc(edge_vec[:, 0], edge_vec[:, 1], edge_vec[:, 2]).reshape(E, 1)
    grid = E // TM
    out = pl.pallas_call(
        _fused_body,
        grid=(grid,),
        in_specs=[
            pl.BlockSpec((TM, 1), lambda i: (i, 0)),
            pl.BlockSpec((TM, D), lambda i: (i, 0)),
            pl.BlockSpec((NUM_EXPERTS, D, H), lambda i: (0, 0, 0)),
            pl.BlockSpec((NUM_EXPERTS, H), lambda i: (0, 0)),
            pl.BlockSpec((NUM_EXPERTS, H, OO), lambda i: (0, 0, 0)),
            pl.BlockSpec((NUM_EXPERTS, OO), lambda i: (0, 0)),
        ],
        out_specs=pl.BlockSpec((TM, OO), lambda i: (i, 0)),
        out_shape=jax.ShapeDtypeStruct((E, OO), jnp.bfloat16),
        compiler_params=pltpu.CompilerParams(
            dimension_semantics=("arbitrary",),
        ),
    )(route, edge_feat,
      W1.astype(jnp.bfloat16), b1, W2.astype(jnp.bfloat16), b2)
    return out.astype(jnp.float32).reshape(E, ORB, ORB)
